# baseline (speedup 1.0000x reference)
import functools

import jax
import jax.numpy as jnp
from jax import lax
from jax.experimental import pallas as pl
from jax.experimental.pallas import tpu as pltpu
from jax.experimental.pallas import tpu_sc as plsc

BATCH = 4096
HIST = 50
EMBED = 128
NUM_CORES = 2
NUM_SUBCORES = 16
NUM_WORKERS = NUM_CORES * NUM_SUBCORES
COLS_PER_W = BATCH // NUM_WORKERS
NBUF = 6


def _make_emb_kernel():
  mesh = plsc.VectorSubcoreMesh(core_axis_name="c", subcore_axis_name="s")

  @functools.partial(
      pl.kernel,
      mesh=mesh,
      out_type=jax.ShapeDtypeStruct((HIST, BATCH, EMBED), jnp.float32),
      scratch_types=[
          pltpu.VMEM((HIST, COLS_PER_W), jnp.int32),
          pltpu.VMEM((NBUF, COLS_PER_W, EMBED), jnp.float32),
          pltpu.SemaphoreType.DMA,
          pltpu.SemaphoreType.DMA,
      ],
  )
  def emb(idx_hbm, table_hbm, out_hbm, idx_v, rows_v, gsem, osem):
    wid = lax.axis_index("s") * NUM_CORES + lax.axis_index("c")
    col0 = wid * COLS_PER_W
    pltpu.sync_copy(idx_hbm.at[:, pl.ds(col0, COLS_PER_W)], idx_v)

    for p in range(NBUF - 1):
      pltpu.async_copy(table_hbm.at[idx_v.at[p]], rows_v.at[p], gsem)

    def body(g, carry):
      cur = lax.rem(g, NBUF)

      @pl.when(g > 0)
      def _wait_prev_store():
        pltpu.make_async_copy(
            rows_v.at[lax.rem(g + NBUF - 1, NBUF)],
            out_hbm.at[g - 1].at[pl.ds(col0, COLS_PER_W)],
            osem,
        ).wait()

      @pl.when(g + NBUF - 1 < HIST)
      def _fire_next_gather():
        pltpu.async_copy(
            table_hbm.at[idx_v.at[g + NBUF - 1]],
            rows_v.at[lax.rem(g + NBUF - 1, NBUF)], gsem)

      pltpu.make_async_copy(
          table_hbm.at[idx_v.at[g]], rows_v.at[cur], gsem).wait()
      pltpu.async_copy(
          rows_v.at[cur], out_hbm.at[g].at[pl.ds(col0, COLS_PER_W)], osem)
      return carry

    lax.fori_loop(0, HIST, body, 0)
    pltpu.make_async_copy(
        rows_v.at[(HIST - 1) % NBUF],
        out_hbm.at[HIST - 1].at[pl.ds(col0, COLS_PER_W)],
        osem,
    ).wait()

  return emb


_EMB = _make_emb_kernel()


@jax.jit
def kernel(input, table):
  out_t = _EMB(input.T, table)
  return out_t.transpose(1, 0, 2)


# speedup vs baseline: 1.0031x; 1.0031x over previous
import functools

import jax
import jax.numpy as jnp
from jax import lax
from jax.experimental import pallas as pl
from jax.experimental.pallas import tpu as pltpu
from jax.experimental.pallas import tpu_sc as plsc

BATCH = 4096
HIST = 50
EMBED = 128
NUM_CORES = 2
NUM_SUBCORES = 16
NUM_WORKERS = NUM_CORES * NUM_SUBCORES
COLS_PER_W = BATCH // NUM_WORKERS
NBUF = 4


def _make_emb_kernel():
  mesh = plsc.VectorSubcoreMesh(core_axis_name="c", subcore_axis_name="s")

  @functools.partial(
      pl.kernel,
      mesh=mesh,
      compiler_params=pltpu.CompilerParams(skip_device_barrier=True),
      out_type=jax.ShapeDtypeStruct((HIST, BATCH, EMBED), jnp.float32),
      scratch_types=[
          pltpu.VMEM((HIST, COLS_PER_W), jnp.int32),
          pltpu.VMEM((NBUF, COLS_PER_W, EMBED), jnp.float32),
          pltpu.SemaphoreType.DMA,
          pltpu.SemaphoreType.DMA,
      ],
  )
  def emb(idx_hbm, table_hbm, out_hbm, idx_v, rows_v, gsem, osem):
    wid = lax.axis_index("s") * NUM_CORES + lax.axis_index("c")
    col0 = wid * COLS_PER_W
    pltpu.sync_copy(idx_hbm.at[:, pl.ds(col0, COLS_PER_W)], idx_v)

    for p in range(NBUF - 1):
      pltpu.async_copy(table_hbm.at[idx_v.at[p]], rows_v.at[p], gsem)

    def body(g, carry):
      cur = lax.rem(g, NBUF)

      @pl.when(g > 0)
      def _wait_prev_store():
        pltpu.make_async_copy(
            rows_v.at[lax.rem(g + NBUF - 1, NBUF)],
            out_hbm.at[g - 1].at[pl.ds(col0, COLS_PER_W)],
            osem,
        ).wait()

      @pl.when(g + NBUF - 1 < HIST)
      def _fire_next_gather():
        pltpu.async_copy(
            table_hbm.at[idx_v.at[g + NBUF - 1]],
            rows_v.at[lax.rem(g + NBUF - 1, NBUF)], gsem)

      pltpu.make_async_copy(
          table_hbm.at[idx_v.at[g]], rows_v.at[cur], gsem).wait()
      pltpu.async_copy(
          rows_v.at[cur], out_hbm.at[g].at[pl.ds(col0, COLS_PER_W)], osem)
      return carry

    lax.fori_loop(0, HIST, body, 0)
    pltpu.make_async_copy(
        rows_v.at[(HIST - 1) % NBUF],
        out_hbm.at[HIST - 1].at[pl.ds(col0, COLS_PER_W)],
        osem,
    ).wait()

  return emb


_EMB = _make_emb_kernel()


@jax.jit
def kernel(input, table):
  out_t = _EMB(input.T, table)
  return out_t.transpose(1, 0, 2)


# speedup vs baseline: 1.0072x; 1.0041x over previous
import functools

import jax
import jax.numpy as jnp
from jax import lax
from jax.experimental import pallas as pl
from jax.experimental.pallas import tpu as pltpu
from jax.experimental.pallas import tpu_sc as plsc

BATCH = 4096
HIST = 50
EMBED = 128
NUM_CORES = 2
NUM_SUBCORES = 16
NUM_WORKERS = NUM_CORES * NUM_SUBCORES
COLS_PER_W = BATCH // NUM_WORKERS
NBUF = 4


def _make_emb_kernel():
  mesh = plsc.VectorSubcoreMesh(core_axis_name="c", subcore_axis_name="s")

  @functools.partial(
      pl.kernel,
      mesh=mesh,
      out_type=jax.ShapeDtypeStruct((HIST, BATCH, EMBED), jnp.float32),
      scratch_types=[
          pltpu.VMEM((HIST, COLS_PER_W), jnp.int32),
          pltpu.VMEM((NBUF, COLS_PER_W, EMBED), jnp.float32),
          pltpu.SemaphoreType.DMA,
          pltpu.SemaphoreType.DMA,
          pltpu.SemaphoreType.DMA,
      ],
  )
  def emb(idx_hbm, table_hbm, out_hbm, idx_v, rows_v, gsem, osem, isem):
    wid = lax.axis_index("s") * NUM_CORES + lax.axis_index("c")
    col0 = wid * COLS_PER_W
    head = pltpu.async_copy(
        idx_hbm.at[pl.ds(0, 8), pl.ds(col0, COLS_PER_W)],
        idx_v.at[pl.ds(0, 8)], isem)
    tail = pltpu.async_copy(
        idx_hbm.at[pl.ds(8, HIST - 8), pl.ds(col0, COLS_PER_W)],
        idx_v.at[pl.ds(8, HIST - 8)], isem)
    head.wait()

    for p in range(NBUF - 1):
      pltpu.async_copy(table_hbm.at[idx_v.at[p]], rows_v.at[p], gsem)
    tail.wait()

    def body(g, carry):
      cur = lax.rem(g, NBUF)

      @pl.when(g > 0)
      def _wait_prev_store():
        pltpu.make_async_copy(
            rows_v.at[lax.rem(g + NBUF - 1, NBUF)],
            out_hbm.at[g - 1].at[pl.ds(col0, COLS_PER_W)],
            osem,
        ).wait()

      @pl.when(g + NBUF - 1 < HIST)
      def _fire_next_gather():
        pltpu.async_copy(
            table_hbm.at[idx_v.at[g + NBUF - 1]],
            rows_v.at[lax.rem(g + NBUF - 1, NBUF)], gsem)

      pltpu.make_async_copy(
          table_hbm.at[idx_v.at[g]], rows_v.at[cur], gsem).wait()
      pltpu.async_copy(
          rows_v.at[cur], out_hbm.at[g].at[pl.ds(col0, COLS_PER_W)], osem)
      return carry

    lax.fori_loop(0, HIST, body, 0)
    pltpu.make_async_copy(
        rows_v.at[(HIST - 1) % NBUF],
        out_hbm.at[HIST - 1].at[pl.ds(col0, COLS_PER_W)],
        osem,
    ).wait()

  return emb


_EMB = _make_emb_kernel()


@jax.jit
def kernel(input, table):
  out_t = _EMB(input.T, table)
  return out_t.transpose(1, 0, 2)
